# baseline (speedup 1.0000x reference)
import functools

import jax
import jax.numpy as jnp
from jax import lax
from jax.experimental import pallas as pl
from jax.experimental.pallas import tpu as pltpu
from jax.experimental.pallas import tpu_sc as plsc

_DIM = 128
_KQ = 100000
_B = 4096

_NC = 2
_NS = 16
_NW = _NC * _NS
_L = 16
_RPW = _B // _NW
_GPW = _RPW // _L


_FLAG_BLK = 128


def _flags_body(rc, tc, ec, rr, tr, er, fo):
    ri = rc[...]
    ti = tc[...]
    ei = ec[...]
    rj = rr[...]
    tj = tr[...]
    ej = er[...]
    rank_ij = (ti < tj) | ((ti == tj) & (ej == 0.0))
    rowv = rank_ij & (ri < rj)
    rank_ji = (tj < ti) | ((tj == ti) & (ei == 0.0))
    colv = rank_ji & (rj < ri) & (ej != 0.0)
    anyrow = jnp.any(rowv, axis=1, keepdims=True)
    anycol = jnp.any(colv, axis=1, keepdims=True)
    hard = ((ei != 0.0) & anyrow) | anycol
    fo[...] = hard.astype(jnp.int32)


def _compute_flags(risk, t_new, e_new):
    col = lambda x: x.reshape(_B, 1)
    row = lambda x: x.reshape(1, _B)
    grid = _B // _FLAG_BLK
    blk_col = pl.BlockSpec((_FLAG_BLK, 1), lambda i: (i, 0))
    blk_row = pl.BlockSpec((1, _B), lambda i: (0, 0))
    flags = pl.pallas_call(
        _flags_body,
        grid=(grid,),
        in_specs=[blk_col, blk_col, blk_col, blk_row, blk_row, blk_row],
        out_specs=pl.BlockSpec((_FLAG_BLK, 1), lambda i: (i, 0)),
        out_shape=jax.ShapeDtypeStruct((_B, 1), jnp.int32),
    )(col(risk), col(t_new), col(e_new), row(risk), row(t_new), row(e_new))
    return flags.reshape(_B)


_ZBLK = 1000
_E2D = (625, 160)


def _copy_body(z_in, e_in, t_in, b_in, z_out, e_out, t_out, b_out):
    z_out[...] = z_in[...]

    @pl.when(pl.program_id(0) == 0)
    def _():
        e_out[...] = e_in[...]
        t_out[...] = t_in[...]
        b_out[...] = b_in[...]


def _copy_buffers(z, e, t, b):
    grid = _KQ // _ZBLK
    whole = pl.BlockSpec(_E2D, lambda i: (0, 0))
    zc, ec, tc, bc = pl.pallas_call(
        _copy_body,
        grid=(grid,),
        in_specs=[pl.BlockSpec((_ZBLK, _DIM), lambda i: (i, 0)),
                  whole, whole, whole],
        out_specs=[pl.BlockSpec((_ZBLK, _DIM), lambda i: (i, 0)),
                   whole, whole, whole],
        out_shape=[jax.ShapeDtypeStruct((_KQ, _DIM), jnp.float32),
                   jax.ShapeDtypeStruct(_E2D, jnp.float32),
                   jax.ShapeDtypeStruct(_E2D, jnp.float32),
                   jax.ShapeDtypeStruct(_E2D, jnp.int32)],
    )(z, e.reshape(_E2D), t.reshape(_E2D), b.reshape(_E2D))
    return zc, ec.reshape(_KQ), tc.reshape(_KQ), bc.reshape(_KQ)


def _sc_body(flags_h, znew_h, enew_h, tnew_h, bnew_h, p_h, s_h,
             zorig_h, eorig_h, torig_h, borig_h,
             zb, eb, tb, bb,
             ptro, sizo,
             flags_v, dst_v, ridx_v, rows_v, e_v, t_v, b_v,
             rz_v, re_v, rt_v, rb_v, pv_v, sv_v, out_v,
             sem_a, sem_b):
    c = lax.axis_index("c")
    s = lax.axis_index("s")
    wid = c * _NS + s
    base_row = wid * _RPW

    pltpu.sync_copy(flags_h, flags_v)
    pltpu.sync_copy(p_h, pv_v)
    pltpu.sync_copy(s_h, sv_v)
    pvec = pv_v[...]

    def _acc(g, a):
        return a + flags_v[pl.ds(g * _L, _L)]

    accv = lax.fori_loop(0, wid * _GPW, _acc, jnp.zeros((_L,), jnp.int32))
    base0 = jnp.sum(accv)

    lane = lax.iota(jnp.int32, _L)

    def _grp(g, cnt):
        f = flags_v[pl.ds((wid * _GPW + g) * _L, _L)]
        rank = (base0 + cnt + jnp.cumsum(f)) - f
        trash = (pvec + (_B + base_row + g * _L) + lane) % _KQ
        real = (pvec + rank) % _KQ
        dst_v[pl.ds(g * _L, _L)] = jnp.where(f > 0, real, trash)
        ridx_v[pl.ds(g * _L, _L)] = trash
        return cnt + jnp.sum(f)

    mycnt = lax.fori_loop(0, _GPW, _grp, 0)
    del mycnt

    pltpu.sync_copy(znew_h.at[pl.ds(base_row, _RPW)], rows_v)
    pltpu.sync_copy(enew_h.at[pl.ds(base_row, _RPW)], e_v)
    pltpu.sync_copy(tnew_h.at[pl.ds(base_row, _RPW)], t_v)
    pltpu.sync_copy(bnew_h.at[pl.ds(base_row, _RPW)], b_v)

    sz = pltpu.async_copy(rows_v, zb.at[dst_v], sem_a)
    se = pltpu.async_copy(e_v, eb.at[dst_v], sem_a)
    st = pltpu.async_copy(t_v, tb.at[dst_v], sem_a)
    sb = pltpu.async_copy(b_v, bb.at[dst_v], sem_a)
    gz = pltpu.async_copy(zorig_h.at[ridx_v], rz_v, sem_b)
    ge = pltpu.async_copy(eorig_h.at[ridx_v], re_v, sem_b)
    gt = pltpu.async_copy(torig_h.at[ridx_v], rt_v, sem_b)
    gb = pltpu.async_copy(borig_h.at[ridx_v], rb_v, sem_b)
    sz.wait(); se.wait(); st.wait(); sb.wait()
    gz.wait(); ge.wait(); gt.wait(); gb.wait()

    rz = pltpu.async_copy(rz_v, zb.at[ridx_v], sem_a)
    re = pltpu.async_copy(re_v, eb.at[ridx_v], sem_a)
    rt = pltpu.async_copy(rt_v, tb.at[ridx_v], sem_a)
    rb = pltpu.async_copy(rb_v, bb.at[ridx_v], sem_a)
    rz.wait(); re.wait(); rt.wait(); rb.wait()

    @pl.when(wid == 0)
    def _():
        def _tot(g, a):
            return a + flags_v[pl.ds(g * _L, _L)]

        totv = lax.fori_loop(0, _B // _L, _tot, jnp.zeros((_L,), jnp.int32))
        bh = jnp.sum(totv)
        p0 = jnp.max(pvec)
        s0 = jnp.max(sv_v[...])
        out_v[...] = jnp.broadcast_to((p0 + bh) % _KQ, (_L,)).astype(jnp.int32)
        pltpu.sync_copy(out_v.at[pl.ds(0, 1)], ptro)
        out_v[...] = jnp.broadcast_to(jnp.minimum(s0 + bh, _KQ), (_L,)).astype(jnp.int32)
        pltpu.sync_copy(out_v.at[pl.ds(0, 1)], sizo)


def _sc_scatter(flags, z_new, e_new, t_new, b_new, pvec, svec,
                z, e, t, b, z_r, e_r, t_r, b_r):
    mesh = plsc.VectorSubcoreMesh(core_axis_name="c", subcore_axis_name="s",
                                  num_cores=_NC, num_subcores=_NS)
    kern = pl.kernel(
        _sc_body,
        out_type=(jax.ShapeDtypeStruct((1,), jnp.int32),
                  jax.ShapeDtypeStruct((1,), jnp.int32)),
        mesh=mesh,
        compiler_params=pltpu.CompilerParams(needs_layout_passes=False),
        scratch_types=(
            pltpu.VMEM((_B,), jnp.int32),
            pltpu.VMEM((_RPW,), jnp.int32),
            pltpu.VMEM((_RPW,), jnp.int32),
            pltpu.VMEM((_RPW, _DIM), jnp.float32),
            pltpu.VMEM((_RPW,), jnp.float32),
            pltpu.VMEM((_RPW,), jnp.float32),
            pltpu.VMEM((_RPW,), jnp.int32),
            pltpu.VMEM((_RPW, _DIM), jnp.float32),
            pltpu.VMEM((_RPW,), jnp.float32),
            pltpu.VMEM((_RPW,), jnp.float32),
            pltpu.VMEM((_RPW,), jnp.int32),
            pltpu.VMEM((_L,), jnp.int32),
            pltpu.VMEM((_L,), jnp.int32),
            pltpu.VMEM((_L,), jnp.int32),
            pltpu.SemaphoreType.DMA,
            pltpu.SemaphoreType.DMA,
        ),
    )
    return kern(flags, z_new, e_new, t_new, b_new, pvec, svec,
                z, e, t, b, z_r, e_r, t_r, b_r)


def kernel(risk, z_new, e_new, t_new, b_new, z, e, t, b, ptr, size):
    flags = _compute_flags(risk, t_new, e_new)
    zc, ec, tc, bc = _copy_buffers(z, e, t, b)
    z_r = jax.new_ref(zc)
    e_r = jax.new_ref(ec)
    t_r = jax.new_ref(tc)
    b_r = jax.new_ref(bc)
    pvec = jnp.broadcast_to(ptr.astype(jnp.int32), (_L,))
    svec = jnp.broadcast_to(size.astype(jnp.int32), (_L,))
    ptr_o, size_o = _sc_scatter(flags, z_new, e_new, t_new, b_new,
                                pvec, svec, z, e, t, b,
                                z_r, e_r, t_r, b_r)
    return (z_r[...], e_r[...], t_r[...], b_r[...], ptr_o, size_o)
